# Initial kernel scaffold; baseline (speedup 1.0000x reference)
import jax
import jax.numpy as jnp
from jax.experimental import pallas as pl


def kernel(feat, feat_a, adj, graph_neigh, weight1, weight2, disc_w, disc_b):
    raise NotImplementedError("write your pallas kernel here")


# speedup vs baseline: 1.6426x; 1.6426x over previous
import functools

import jax
import jax.numpy as jnp
from jax.experimental import pallas as pl

N = 4096
IN_F = 256
OUT_F = 64
BM = 256


def _k_m1(feat_ref, feata_ref, w1_ref, m1_ref):
    w1 = w1_ref[...]
    m1_ref[:, :OUT_F] = jnp.dot(feat_ref[...], w1, preferred_element_type=jnp.float32)
    m1_ref[:, OUT_F:] = jnp.dot(feata_ref[...], w1, preferred_element_type=jnp.float32)


def _k_z(adj_ref, m1_ref, w2_ref, zc_ref, h2_ref):
    z = jnp.dot(adj_ref[...], m1_ref[...], preferred_element_type=jnp.float32)
    zc_ref[...] = z
    h2_ref[...] = jnp.dot(z[:, :OUT_F], w2_ref[...], preferred_element_type=jnp.float32)


def _l2norm_sigmoid(x):
    n = jnp.sqrt(jnp.sum(x * x, axis=1, keepdims=True))
    return jax.nn.sigmoid(x / jnp.maximum(n, 1e-12))


def _k_main(adj_ref, gn_ref, h2_ref, zc_ref, dw_ref, db_ref,
            h_ref, ret_ref, reta_ref):
    i = pl.program_id(0)
    adj = adj_ref[...]
    gn = gn_ref[...]

    h_ref[...] = jnp.dot(adj, h2_ref[...], preferred_element_type=jnp.float32)

    emb_full = jnp.maximum(zc_ref[...], 0.0)
    vsum = jnp.dot(gn, emb_full, preferred_element_type=jnp.float32)
    rowsum = jnp.sum(gn, axis=1, keepdims=True)
    g = _l2norm_sigmoid(vsum[:, :OUT_F] / rowsum)
    ga = _l2norm_sigmoid(vsum[:, OUT_F:] / rowsum)

    zblk = zc_ref[pl.ds(i * BM, BM), :]
    emb = jnp.maximum(zblk[:, :OUT_F], 0.0)
    emba = jnp.maximum(zblk[:, OUT_F:], 0.0)
    dw = dw_ref[...]
    t = jnp.dot(emb, dw, preferred_element_type=jnp.float32)
    ta = jnp.dot(emba, dw, preferred_element_type=jnp.float32)
    b = db_ref[0, 0]
    sc1 = jnp.sum(t * g, axis=1, keepdims=True) + b
    sc2 = jnp.sum(ta * g, axis=1, keepdims=True) + b
    ret_ref[...] = jnp.concatenate([sc1, sc2], axis=1)
    sc1a = jnp.sum(ta * ga, axis=1, keepdims=True) + b
    sc2a = jnp.sum(t * ga, axis=1, keepdims=True) + b
    reta_ref[...] = jnp.concatenate([sc1a, sc2a], axis=1)


@functools.partial(jax.jit, static_argnames=("interpret",))
def kernel(feat, feat_a, adj, graph_neigh, weight1, weight2, disc_w, disc_b,
           interpret=False):
    f32 = jnp.float32
    m1 = pl.pallas_call(
        _k_m1,
        out_shape=jax.ShapeDtypeStruct((N, 2 * OUT_F), f32),
        interpret=interpret,
    )(feat, feat_a, weight1)

    zcat, h2 = pl.pallas_call(
        _k_z,
        grid=(N // BM,),
        in_specs=[
            pl.BlockSpec((BM, N), lambda i: (i, 0)),
            pl.BlockSpec((N, 2 * OUT_F), lambda i: (0, 0)),
            pl.BlockSpec((OUT_F, IN_F), lambda i: (0, 0)),
        ],
        out_specs=[
            pl.BlockSpec((BM, 2 * OUT_F), lambda i: (i, 0)),
            pl.BlockSpec((BM, IN_F), lambda i: (i, 0)),
        ],
        out_shape=[
            jax.ShapeDtypeStruct((N, 2 * OUT_F), f32),
            jax.ShapeDtypeStruct((N, IN_F), f32),
        ],
        interpret=interpret,
    )(adj, m1, weight2)

    h, ret, ret_a = pl.pallas_call(
        _k_main,
        grid=(N // BM,),
        in_specs=[
            pl.BlockSpec((BM, N), lambda i: (i, 0)),
            pl.BlockSpec((BM, N), lambda i: (i, 0)),
            pl.BlockSpec((N, IN_F), lambda i: (0, 0)),
            pl.BlockSpec((N, 2 * OUT_F), lambda i: (0, 0)),
            pl.BlockSpec((OUT_F, OUT_F), lambda i: (0, 0)),
            pl.BlockSpec((1, 1), lambda i: (0, 0)),
        ],
        out_specs=[
            pl.BlockSpec((BM, IN_F), lambda i: (i, 0)),
            pl.BlockSpec((BM, 2), lambda i: (i, 0)),
            pl.BlockSpec((BM, 2), lambda i: (i, 0)),
        ],
        out_shape=[
            jax.ShapeDtypeStruct((N, IN_F), f32),
            jax.ShapeDtypeStruct((N, 2), f32),
            jax.ShapeDtypeStruct((N, 2), f32),
        ],
        interpret=interpret,
    )(adj, graph_neigh, h2, zcat, disc_w, disc_b.reshape(1, 1))

    hiden_emb = zcat[:, :OUT_F]
    clustering_loss = jnp.zeros((), f32)
    return (hiden_emb, h, ret, ret_a, clustering_loss)
